# Initial kernel scaffold; baseline (speedup 1.0000x reference)
import jax
import jax.numpy as jnp
from jax import lax
from jax.experimental import pallas as pl
from jax.experimental.pallas import tpu as pltpu
from jax.experimental.pallas import tpu_sc as plsc

N = 10000
E = 320000
D = 128
NC = 2
NS = 16
NT = NC * NS
EPT = E // NT
CHUNK = 125
NCHUNK = EPT // CHUNK
RPT = N // NS
EPS = 1e-3


def _deg_body(row_hbm, ew_hbm, out_hbm, row_v, ew_v, deg_v):
    c = lax.axis_index("c")
    s = lax.axis_index("s")
    wid = c * NS + s
    base = wid * EPT
    pltpu.sync_copy(row_hbm.at[pl.ds(base, EPT)], row_v)
    pltpu.sync_copy(ew_hbm.at[pl.ds(base, EPT)], ew_v)
    z16 = jnp.zeros((16,), jnp.float32)

    @pl.loop(0, N, step=16)
    def _(i):
        deg_v[pl.ds(i, 16)] = z16

    @pl.loop(0, EPT, step=16)
    def _(i):
        idx = row_v[pl.ds(i, 16)]
        vals = ew_v[pl.ds(i, 16)]
        plsc.addupdate_scatter(deg_v, [idx], vals)

    pltpu.sync_copy(deg_v, out_hbm.at[wid])


def _prep_body(pt_ref, x_ref, dis_ref, g_ref):
    deg = jnp.sum(pt_ref[...], axis=1, keepdims=True) + 1.0
    dis = lax.rsqrt(deg)
    dis_ref[...] = dis
    g_ref[...] = x_ref[...] * dis


def _agg_body(g_hbm, row_hbm, col_hbm, ew_hbm, out_hbm,
              row_v, col_v, ew_v, rows_v, acc_sh):
    c = lax.axis_index("c")
    s = lax.axis_index("s")
    wid = c * NS + s
    pltpu.sync_copy(row_hbm.at[wid], row_v)
    pltpu.sync_copy(col_hbm.at[wid], col_v)
    pltpu.sync_copy(ew_hbm.at[wid], ew_v)

    z16 = jnp.zeros((16,), jnp.float32)

    @pl.loop(0, CHUNK)
    def _(e):
        for k in range(8):
            rows_v[e, pl.ds(k * 16, 16)] = z16

    @pl.loop(0, RPT, step=CHUNK)
    def _(i):
        pltpu.sync_copy(rows_v, acc_sh.at[pl.ds(s * RPT + i, CHUNK)])

    plsc.subcore_barrier()

    @pl.loop(0, NCHUNK)
    def _(j):
        pltpu.sync_copy(g_hbm.at[col_v.at[j]], rows_v)

        @pl.loop(0, CHUNK)
        def _(e):
            w = ew_v[j, e]
            for k in range(8):
                sl = (e, pl.ds(k * 16, 16))
                rows_v[sl] = rows_v[sl] * w

        pltpu.sync_copy(rows_v, acc_sh.at[row_v.at[j]], add=True)

    plsc.subcore_barrier()

    @pl.loop(0, RPT, step=CHUNK)
    def _(i):
        sl = pl.ds(s * RPT + i, CHUNK)
        pltpu.sync_copy(acc_sh.at[sl], out_hbm.at[c, sl])


def _final_body(acc_ref, g_ref, dis_ref, w_ref, gamma_ref, beta_ref,
                mean_ref, var_ref, bias_ref, o_ref):
    pre = dis_ref[...] * (acc_ref[0] + acc_ref[1] + g_ref[...])
    z = jnp.dot(pre, w_ref[...], preferred_element_type=jnp.float32)
    scale = gamma_ref[...] * lax.rsqrt(var_ref[...] + EPS)
    shift = (bias_ref[...] - mean_ref[...]) * scale + beta_ref[...]
    o_ref[...] = jnp.maximum(z * scale + shift, 0.0)


@jax.jit
def kernel(x, edge_index, edge_weight, W, bias, bn_gamma, bn_beta,
           bn_mean, bn_var):
    row = edge_index[0]
    col = edge_index[1]
    mesh = plsc.VectorSubcoreMesh(core_axis_name="c", subcore_axis_name="s")

    deg_call = pl.kernel(
        _deg_body,
        out_type=jax.ShapeDtypeStruct((NT, N), jnp.float32),
        mesh=mesh,
        scratch_types=[
            pltpu.VMEM((EPT,), jnp.int32),
            pltpu.VMEM((EPT,), jnp.float32),
            pltpu.VMEM((N,), jnp.float32),
        ],
    )
    partials = deg_call(row, edge_weight)

    dis2d, g = pl.pallas_call(
        _prep_body,
        out_shape=[
            jax.ShapeDtypeStruct((N, 1), jnp.float32),
            jax.ShapeDtypeStruct((N, D), jnp.float32),
        ],
    )(partials.T, x)

    row3 = row.reshape(NT, NCHUNK, CHUNK)
    col3 = col.reshape(NT, NCHUNK, CHUNK)
    ew3 = edge_weight.reshape(NT, NCHUNK, CHUNK)

    agg_call = pl.kernel(
        _agg_body,
        out_type=jax.ShapeDtypeStruct((NC, N, D), jnp.float32),
        mesh=mesh,
        scratch_types=[
            pltpu.VMEM((NCHUNK, CHUNK), jnp.int32),
            pltpu.VMEM((NCHUNK, CHUNK), jnp.int32),
            pltpu.VMEM((NCHUNK, CHUNK), jnp.float32),
            pltpu.VMEM((CHUNK, D), jnp.float32),
            pltpu.VMEM_SHARED((N, D), jnp.float32),
        ],
    )
    acc = agg_call(g, row3, col3, ew3)

    R = 1000
    out = pl.pallas_call(
        _final_body,
        grid=(N // R,),
        in_specs=[
            pl.BlockSpec((NC, R, D), lambda i: (0, i, 0)),
            pl.BlockSpec((R, D), lambda i: (i, 0)),
            pl.BlockSpec((R, 1), lambda i: (i, 0)),
            pl.BlockSpec((D, D), lambda i: (0, 0)),
            pl.BlockSpec((1, D), lambda i: (0, 0)),
            pl.BlockSpec((1, D), lambda i: (0, 0)),
            pl.BlockSpec((1, D), lambda i: (0, 0)),
            pl.BlockSpec((1, D), lambda i: (0, 0)),
            pl.BlockSpec((1, D), lambda i: (0, 0)),
        ],
        out_specs=pl.BlockSpec((R, D), lambda i: (i, 0)),
        out_shape=jax.ShapeDtypeStruct((N, D), jnp.float32),
    )(acc, g, dis2d, W,
      bn_gamma.reshape(1, D), bn_beta.reshape(1, D),
      bn_mean.reshape(1, D), bn_var.reshape(1, D), bias.reshape(1, D))
    return out


# speedup vs baseline: 21.6981x; 21.6981x over previous
import jax
import jax.numpy as jnp
from jax import lax
from jax.experimental import pallas as pl
from jax.experimental.pallas import tpu as pltpu
from jax.experimental.pallas import tpu_sc as plsc

N = 10000
E = 320000
D = 128
NC = 2
NS = 16
NT = NC * NS
EPT = E // NT
CHUNK = 80
NCHUNK = EPT // CHUNK
SB = 5
NSB = NCHUNK // SB
FCH = 80
NFCH = N // FCH
EPS = 1e-3


def _deg_body(row_hbm, ew_hbm, out_hbm, row_v, ew_v, deg_v):
    c = lax.axis_index("c")
    s = lax.axis_index("s")
    wid = c * NS + s
    base = wid * EPT
    pltpu.sync_copy(row_hbm.at[pl.ds(base, EPT)], row_v)
    pltpu.sync_copy(ew_hbm.at[pl.ds(base, EPT)], ew_v)
    z16 = jnp.zeros((16,), jnp.float32)

    @pl.loop(0, N, step=16)
    def _(i):
        deg_v[pl.ds(i, 16)] = z16

    @pl.loop(0, EPT, step=16)
    def _(i):
        idx = row_v[pl.ds(i, 16)]
        vals = ew_v[pl.ds(i, 16)]
        plsc.addupdate_scatter(deg_v, [idx], vals)

    pltpu.sync_copy(deg_v, out_hbm.at[wid])


def _prep_body(pt_ref, x_ref, dis_ref, g_ref):
    deg = jnp.sum(pt_ref[...], axis=1, keepdims=True) + 1.0
    dis = lax.rsqrt(deg)
    dis_ref[...] = dis
    g_ref[...] = x_ref[...] * dis


def _agg_body(g_hbm, row_hbm, col_hbm, ew_hbm, out_hbm,
              row_v, col_v, ew_v, rows_v, acc_sh):
    c = lax.axis_index("c")
    s = lax.axis_index("s")
    wid = c * NS + s

    z16 = jnp.zeros((16,), jnp.float32)

    @pl.loop(0, CHUNK)
    def _(e):
        for k in range(D // 16):
            rows_v[e, pl.ds(k * 16, 16)] = z16

    @pl.loop(s, NFCH, step=NS)
    def _(i):
        pltpu.sync_copy(rows_v, acc_sh.at[pl.ds(i * FCH, FCH)])

    plsc.subcore_barrier()

    @pl.loop(0, NSB)
    def _(b):
        pltpu.sync_copy(row_hbm.at[wid, b], row_v)
        pltpu.sync_copy(col_hbm.at[wid, b], col_v)
        pltpu.sync_copy(ew_hbm.at[wid, b], ew_v)

        for j in range(SB):
            pltpu.sync_copy(g_hbm.at[col_v.at[j]], rows_v)

            @pl.loop(0, CHUNK, step=16)
            def _(e):
                w16 = ew_v[j, pl.ds(e, 16)]
                for l in range(16):
                    w = w16[l]
                    for k in range(D // 16):
                        sl = (e + l, pl.ds(k * 16, 16))
                        rows_v[sl] = rows_v[sl] * w

            pltpu.sync_copy(rows_v, acc_sh.at[row_v.at[j]], add=True)

    plsc.subcore_barrier()

    @pl.loop(s, NFCH, step=NS)
    def _(i):
        sl = pl.ds(i * FCH, FCH)
        pltpu.sync_copy(acc_sh.at[sl], out_hbm.at[c, sl])


def _final_body(acc_ref, g_ref, dis_ref, w_ref, gamma_ref, beta_ref,
                mean_ref, var_ref, bias_ref, o_ref):
    pre = dis_ref[...] * (acc_ref[0] + acc_ref[1] + g_ref[...])
    z = jnp.dot(pre, w_ref[...], preferred_element_type=jnp.float32)
    scale = gamma_ref[...] * lax.rsqrt(var_ref[...] + EPS)
    shift = (bias_ref[...] - mean_ref[...]) * scale + beta_ref[...]
    o_ref[...] = jnp.maximum(z * scale + shift, 0.0)


@jax.jit
def kernel(x, edge_index, edge_weight, W, bias, bn_gamma, bn_beta,
           bn_mean, bn_var):
    row = edge_index[0]
    col = edge_index[1]
    mesh = plsc.VectorSubcoreMesh(core_axis_name="c", subcore_axis_name="s")
    sc_params = pltpu.CompilerParams(needs_layout_passes=False)

    deg_call = pl.kernel(
        _deg_body,
        out_type=jax.ShapeDtypeStruct((NT, N), jnp.float32),
        mesh=mesh,
        compiler_params=sc_params,
        scratch_types=[
            pltpu.VMEM((EPT,), jnp.int32),
            pltpu.VMEM((EPT,), jnp.float32),
            pltpu.VMEM((N,), jnp.float32),
        ],
    )
    partials = deg_call(row, edge_weight)

    dis2d, g = pl.pallas_call(
        _prep_body,
        out_shape=[
            jax.ShapeDtypeStruct((N, 1), jnp.float32),
            jax.ShapeDtypeStruct((N, D), jnp.float32),
        ],
    )(partials.T, x)

    row4 = row.reshape(NT, NSB, SB, CHUNK)
    col4 = col.reshape(NT, NSB, SB, CHUNK)
    ew4 = edge_weight.reshape(NT, NSB, SB, CHUNK)

    agg_call = pl.kernel(
        _agg_body,
        out_type=jax.ShapeDtypeStruct((NC, N, D), jnp.float32),
        mesh=mesh,
        compiler_params=sc_params,
        scratch_types=[
            pltpu.VMEM((SB, CHUNK), jnp.int32),
            pltpu.VMEM((SB, CHUNK), jnp.int32),
            pltpu.VMEM((SB, CHUNK), jnp.float32),
            pltpu.VMEM((CHUNK, D), jnp.float32),
            pltpu.VMEM_SHARED((N, D), jnp.float32),
        ],
    )
    acc = agg_call(g, row4, col4, ew4)

    R = 1000
    out = pl.pallas_call(
        _final_body,
        grid=(N // R,),
        in_specs=[
            pl.BlockSpec((NC, R, D), lambda i: (0, i, 0)),
            pl.BlockSpec((R, D), lambda i: (i, 0)),
            pl.BlockSpec((R, 1), lambda i: (i, 0)),
            pl.BlockSpec((D, D), lambda i: (0, 0)),
            pl.BlockSpec((1, D), lambda i: (0, 0)),
            pl.BlockSpec((1, D), lambda i: (0, 0)),
            pl.BlockSpec((1, D), lambda i: (0, 0)),
            pl.BlockSpec((1, D), lambda i: (0, 0)),
            pl.BlockSpec((1, D), lambda i: (0, 0)),
        ],
        out_specs=pl.BlockSpec((R, D), lambda i: (i, 0)),
        out_shape=jax.ShapeDtypeStruct((N, D), jnp.float32),
    )(acc, g, dis2d, W,
      bn_gamma.reshape(1, D), bn_beta.reshape(1, D),
      bn_mean.reshape(1, D), bn_var.reshape(1, D), bias.reshape(1, D))
    return out


# speedup vs baseline: 35.3588x; 1.6296x over previous
import jax
import jax.numpy as jnp
from jax import lax
from jax.experimental import pallas as pl
from jax.experimental.pallas import tpu as pltpu
from jax.experimental.pallas import tpu_sc as plsc

N = 10000
E = 320000
D = 128
NC = 2
NS = 16
NT = NC * NS
EPT = E // NT
CHUNK = 80
NCHUNK = EPT // CHUNK
SBC = 25
NSS = NCHUNK // SBC
FCH = 80
NFCH = N // FCH
EPS = 1e-3


def _deg_body(row_hbm, ew_hbm, out_hbm, row_v, ew_v, deg_v):
    c = lax.axis_index("c")
    s = lax.axis_index("s")
    wid = c * NS + s
    base = wid * EPT
    pltpu.sync_copy(row_hbm.at[pl.ds(base, EPT)], row_v)
    pltpu.sync_copy(ew_hbm.at[pl.ds(base, EPT)], ew_v)
    z16 = jnp.zeros((16,), jnp.float32)

    @pl.loop(0, N, step=16)
    def _(i):
        deg_v[pl.ds(i, 16)] = z16

    @pl.loop(0, EPT, step=16)
    def _(i):
        idx = row_v[pl.ds(i, 16)]
        vals = ew_v[pl.ds(i, 16)]
        plsc.addupdate_scatter(deg_v, [idx], vals)

    pltpu.sync_copy(deg_v, out_hbm.at[wid])


def _prep_body(pt_ref, x_ref, dis_ref, g_ref):
    deg = jnp.sum(pt_ref[...], axis=1, keepdims=True) + 1.0
    dis = lax.rsqrt(deg)
    dis_ref[...] = dis
    g_ref[...] = x_ref[...] * dis


def _agg_body(g_hbm, row_hbm, col_hbm, ew_hbm, out_hbm,
              row_v, col_v, ew_v, rows0, rows1, acc_sh,
              gsem0, gsem1, ssem0, ssem1):
    c = lax.axis_index("c")
    s = lax.axis_index("s")
    wid = c * NS + s

    z16 = jnp.zeros((16,), jnp.float32)

    @pl.loop(0, CHUNK)
    def _(e):
        for k in range(D // 16):
            rows0[e, pl.ds(k * 16, 16)] = z16

    @pl.loop(s, NFCH, step=NS)
    def _(i):
        pltpu.sync_copy(rows0, acc_sh.at[pl.ds(i * FCH, FCH)])

    plsc.subcore_barrier()

    def wait_scat(sem):
        pltpu.make_async_copy(rows0, acc_sh.at[pl.ds(0, CHUNK)], sem).wait()

    def wait_gath(sem, buf):
        pltpu.make_async_copy(g_hbm.at[pl.ds(0, CHUNK)], buf, sem).wait()

    def scale(buf, k):
        @pl.loop(0, CHUNK, step=16)
        def _(e):
            w16 = ew_v[k, pl.ds(e, 16)]
            for l in range(16):
                w = w16[l]
                for kk in range(D // 16):
                    sl = (e + l, pl.ds(kk * 16, 16))
                    buf[sl] = buf[sl] * w

    @pl.loop(0, NSS)
    def _(ss):
        @pl.when(ss > 0)
        def _():
            wait_scat(ssem0)
            wait_scat(ssem1)

        pltpu.sync_copy(row_hbm.at[wid, ss], row_v)
        pltpu.sync_copy(col_hbm.at[wid, ss], col_v)
        pltpu.sync_copy(ew_hbm.at[wid, ss], ew_v)

        pltpu.async_copy(g_hbm.at[col_v.at[0]], rows0, gsem0)

        @pl.loop(1, SBC - 1, step=2)
        def _(k):
            @pl.when(k > 1)
            def _():
                wait_scat(ssem1)
            pltpu.async_copy(g_hbm.at[col_v.at[k]], rows1, gsem1)
            wait_gath(gsem0, rows0)
            scale(rows0, k - 1)
            pltpu.async_copy(rows0, acc_sh.at[row_v.at[k - 1]], ssem0,
                             add=True)
            wait_scat(ssem0)
            pltpu.async_copy(g_hbm.at[col_v.at[k + 1]], rows0, gsem0)
            wait_gath(gsem1, rows1)
            scale(rows1, k)
            pltpu.async_copy(rows1, acc_sh.at[row_v.at[k]], ssem1, add=True)

        wait_gath(gsem0, rows0)
        scale(rows0, SBC - 1)
        pltpu.async_copy(rows0, acc_sh.at[row_v.at[SBC - 1]], ssem0, add=True)

    wait_scat(ssem0)
    wait_scat(ssem1)
    plsc.subcore_barrier()

    @pl.loop(s, NFCH, step=NS)
    def _(i):
        sl = pl.ds(i * FCH, FCH)
        pltpu.sync_copy(acc_sh.at[sl], out_hbm.at[c, sl])


def _final_body(acc_ref, g_ref, dis_ref, w_ref, gamma_ref, beta_ref,
                mean_ref, var_ref, bias_ref, o_ref):
    pre = dis_ref[...] * (acc_ref[0] + acc_ref[1] + g_ref[...])
    z = jnp.dot(pre, w_ref[...], preferred_element_type=jnp.float32)
    scale = gamma_ref[...] * lax.rsqrt(var_ref[...] + EPS)
    shift = (bias_ref[...] - mean_ref[...]) * scale + beta_ref[...]
    o_ref[...] = jnp.maximum(z * scale + shift, 0.0)


@jax.jit
def kernel(x, edge_index, edge_weight, W, bias, bn_gamma, bn_beta,
           bn_mean, bn_var):
    row = edge_index[0]
    col = edge_index[1]
    mesh = plsc.VectorSubcoreMesh(core_axis_name="c", subcore_axis_name="s")
    sc_params = pltpu.CompilerParams(needs_layout_passes=False)

    deg_call = pl.kernel(
        _deg_body,
        out_type=jax.ShapeDtypeStruct((NT, N), jnp.float32),
        mesh=mesh,
        compiler_params=sc_params,
        scratch_types=[
            pltpu.VMEM((EPT,), jnp.int32),
            pltpu.VMEM((EPT,), jnp.float32),
            pltpu.VMEM((N,), jnp.float32),
        ],
    )
    partials = deg_call(row, edge_weight)

    dis2d, g = pl.pallas_call(
        _prep_body,
        out_shape=[
            jax.ShapeDtypeStruct((N, 1), jnp.float32),
            jax.ShapeDtypeStruct((N, D), jnp.float32),
        ],
    )(partials.T, x)

    row4 = row.reshape(NT, NSS, SBC, CHUNK)
    col4 = col.reshape(NT, NSS, SBC, CHUNK)
    ew4 = edge_weight.reshape(NT, NSS, SBC, CHUNK)

    agg_call = pl.kernel(
        _agg_body,
        out_type=jax.ShapeDtypeStruct((NC, N, D), jnp.float32),
        mesh=mesh,
        compiler_params=sc_params,
        scratch_types=[
            pltpu.VMEM((SBC, CHUNK), jnp.int32),
            pltpu.VMEM((SBC, CHUNK), jnp.int32),
            pltpu.VMEM((SBC, CHUNK), jnp.float32),
            pltpu.VMEM((CHUNK, D), jnp.float32),
            pltpu.VMEM((CHUNK, D), jnp.float32),
            pltpu.VMEM_SHARED((N, D), jnp.float32),
            pltpu.SemaphoreType.DMA,
            pltpu.SemaphoreType.DMA,
            pltpu.SemaphoreType.DMA,
            pltpu.SemaphoreType.DMA,
        ],
    )
    acc = agg_call(g, row4, col4, ew4)

    R = 1000
    out = pl.pallas_call(
        _final_body,
        grid=(N // R,),
        in_specs=[
            pl.BlockSpec((NC, R, D), lambda i: (0, i, 0)),
            pl.BlockSpec((R, D), lambda i: (i, 0)),
            pl.BlockSpec((R, 1), lambda i: (i, 0)),
            pl.BlockSpec((D, D), lambda i: (0, 0)),
            pl.BlockSpec((1, D), lambda i: (0, 0)),
            pl.BlockSpec((1, D), lambda i: (0, 0)),
            pl.BlockSpec((1, D), lambda i: (0, 0)),
            pl.BlockSpec((1, D), lambda i: (0, 0)),
            pl.BlockSpec((1, D), lambda i: (0, 0)),
        ],
        out_specs=pl.BlockSpec((R, D), lambda i: (i, 0)),
        out_shape=jax.ShapeDtypeStruct((N, D), jnp.float32),
    )(acc, g, dis2d, W,
      bn_gamma.reshape(1, D), bn_beta.reshape(1, D),
      bn_mean.reshape(1, D), bn_var.reshape(1, D), bias.reshape(1, D))
    return out
